# Initial kernel scaffold; baseline (speedup 1.0000x reference)
import jax
import jax.numpy as jnp
from jax.experimental import pallas as pl


def kernel(x, edge_index, edge_weight, W0_z, W1_z, b_z, W0_r, W1_r, b_r, W0_h, W1_h, b_h):
    raise NotImplementedError("write your pallas kernel here")


# speedup vs baseline: 9.8616x; 9.8616x over previous
import functools

import jax
import jax.numpy as jnp
from jax import lax
from jax.experimental import pallas as pl
from jax.experimental.pallas import tpu as pltpu
from jax.experimental.pallas import tpu_sc as plsc

N = 10000
F = 128
E = 320000

NC = 2
NS = 16
L = 16
NW = NC * NS
EPW = E // NW
B = 80
NCHUNK = EPW // B
NZCH = N // B


def _sc_body(x_hbm, src_hbm, dst_hbm, w_hbm, agg_out, deg_out,
             src_v, dst_v, w_v, rows_v, zrow_v, zdeg_v, acc_s, deg_s, sem):
    cid = lax.axis_index("c")
    sid = lax.axis_index("s")
    wid = sid * NC + cid

    zero16 = jnp.zeros((L,), jnp.float32)

    def _zrow(i, carry):
        for j in range(F // L):
            zrow_v[i, pl.ds(j * L, L)] = zero16
        return carry
    lax.fori_loop(0, B, _zrow, 0)
    for j in range(B // L):
        zdeg_v[pl.ds(j * L, L)] = zero16

    for k in range(-(-NZCH // NS)):
        c = sid + k * NS

        @pl.when(c < NZCH)
        def _():
            pltpu.sync_copy(zrow_v, acc_s.at[pl.ds(c * B, B)])
            pltpu.sync_copy(zdeg_v, deg_s.at[pl.ds(c * B, B)])

    plsc.subcore_barrier()

    ebase = wid * EPW

    def _chunk(c, carry):
        off = ebase + c * B
        pltpu.sync_copy(src_hbm.at[pl.ds(off, B)], src_v)
        pltpu.sync_copy(dst_hbm.at[pl.ds(off, B)], dst_v)
        pltpu.sync_copy(w_hbm.at[pl.ds(off, B)], w_v)
        pltpu.async_copy(x_hbm.at[src_v], rows_v, sem).wait()

        pltpu.sync_copy(w_v, deg_s.at[dst_v], add=True)

        def _grp(g, carry2):
            for k2 in range(L):
                e = g * L + k2
                wb = plsc.load_gather(
                    w_v, [jnp.full((L,), e, jnp.int32)])
                for j in range(F // L):
                    rows_v[e, pl.ds(j * L, L)] = (
                        rows_v[e, pl.ds(j * L, L)] * wb)
            return carry2
        lax.fori_loop(0, B // L, _grp, 0)

        pltpu.sync_copy(rows_v, acc_s.at[dst_v], add=True)
        return carry
    lax.fori_loop(0, NCHUNK, _chunk, 0)

    plsc.subcore_barrier()

    @pl.when(sid == 0)
    def _():
        pltpu.sync_copy(acc_s, agg_out.at[cid])
        pltpu.sync_copy(deg_s, deg_out.at[cid])


_sc_agg = functools.partial(
    pl.kernel,
    out_type=(
        jax.ShapeDtypeStruct((NC, N, F), jnp.float32),
        jax.ShapeDtypeStruct((NC, N), jnp.float32),
    ),
    mesh=plsc.VectorSubcoreMesh(core_axis_name="c", subcore_axis_name="s"),
    compiler_params=pltpu.CompilerParams(needs_layout_passes=False),
    scratch_types=[
        pltpu.VMEM((B,), jnp.int32),
        pltpu.VMEM((B,), jnp.int32),
        pltpu.VMEM((B,), jnp.float32),
        pltpu.VMEM((B, F), jnp.float32),
        pltpu.VMEM((B, F), jnp.float32),
        pltpu.VMEM((B,), jnp.float32),
        pltpu.VMEM_SHARED((N, F), jnp.float32),
        pltpu.VMEM_SHARED((N,), jnp.float32),
        pltpu.SemaphoreType.DMA,
    ],
)(_sc_body)


RB = 1000


def _tc_body(x_ref, a0_ref, a1_ref, d0_ref, d1_ref,
             az_ref, bz_ref, ah_ref, bh_ref, vz_ref, vh_ref, o_ref):
    x = x_ref[...]
    agg = a0_ref[...] + a1_ref[...]
    deg = d0_ref[...] + d1_ref[...]
    deg_inv = jnp.where(deg > 0, 1.0 / deg, 0.0)
    agg = agg * deg_inv
    pz = (jnp.dot(x, az_ref[...], preferred_element_type=jnp.float32)
          + jnp.dot(agg, bz_ref[...], preferred_element_type=jnp.float32)
          + vz_ref[...])
    ph = (jnp.dot(x, ah_ref[...], preferred_element_type=jnp.float32)
          + jnp.dot(agg, bh_ref[...], preferred_element_type=jnp.float32)
          + vh_ref[...])
    z = jax.nn.sigmoid(pz)
    ht = jnp.tanh(ph)
    o_ref[...] = jnp.maximum((1.0 - z) * ht, 0.0)


def _tc_gru(x, a0, a1, d0, d1, az, bz, ah, bh, vz, vh):
    grid = (N // RB,)
    row = pl.BlockSpec((RB, F), lambda i: (i, 0))
    col = pl.BlockSpec((RB, 1), lambda i: (i, 0))
    full = pl.BlockSpec((F, F), lambda i: (0, 0))
    vec = pl.BlockSpec((1, F), lambda i: (0, 0))
    return pl.pallas_call(
        _tc_body,
        grid=grid,
        in_specs=[row, row, row, col, col, full, full, full, full, vec, vec],
        out_specs=row,
        out_shape=jax.ShapeDtypeStruct((N, F), jnp.float32),
    )(x, a0, a1, d0, d1, az, bz, ah, bh, vz, vh)


def kernel(x, edge_index, edge_weight,
           W0_z, W1_z, b_z, W0_r, W1_r, b_r, W0_h, W1_h, b_h):
    src = edge_index[0]
    dst = edge_index[1]
    agg_parts, deg_parts = _sc_agg(x, src, dst, edge_weight)
    return _tc_gru(
        x, agg_parts[0], agg_parts[1],
        deg_parts[0][:, None], deg_parts[1][:, None],
        W0_z[:F], W1_z[:F], W0_h[:F], W1_h[:F],
        b_z[None, :], b_h[None, :])
